# Initial kernel scaffold; baseline (speedup 1.0000x reference)
import jax
import jax.numpy as jnp
from jax.experimental import pallas as pl


def kernel(x, W0, b0, W1, b1, W2, b2):
    raise NotImplementedError("write your pallas kernel here")


# speedup vs baseline: 1.0099x; 1.0099x over previous
import math

import jax
import jax.numpy as jnp
import numpy as np
from jax import lax
from jax.experimental import pallas as pl
from jax.experimental.pallas import tpu as pltpu

_NUM_BINS = 8
_NCH = 16
_GL = _NCH * _NUM_BINS
_LEFT = -1.0
_SPAN = 2.0
_MINW = 1e-4
_MINH = 1e-4
_MIND = 1e-4
_BLIM = _LEFT + 1e-3
_ULIM = -_LEFT - 1e-3
_DCONST = math.log(math.exp(1.0 - _MIND) - 1.0)


def _softplus(z):
    return jnp.maximum(z, 0.0) + jnp.log(1.0 + jnp.exp(-jnp.abs(z)))


def _spline_body(x_ref, w0_ref, b0_ref, w1_ref, b1_ref,
                 w2w_ref, b2w_ref, w2h_ref, b2h_ref, w2d_ref, b2d_ref,
                 e1t_ref, e2t_ref, e1_ref, e2_ref, bb_ref, u_ref, p_ref,
                 out_ref, ld_ref):
    f32 = jnp.float32
    x = x_ref[...]
    x1 = jnp.dot(x, e1t_ref[...], preferred_element_type=f32, precision=lax.Precision.HIGHEST)
    x2 = jnp.dot(x, e2t_ref[...], preferred_element_type=f32, precision=lax.Precision.HIGHEST)

    h = jnp.dot(x2, w0_ref[...], preferred_element_type=f32, precision=lax.Precision.HIGHEST) + b0_ref[...]
    h = jnp.maximum(h, 0.0)
    h = jnp.dot(h, w1_ref[...], preferred_element_type=f32, precision=lax.Precision.HIGHEST) + b1_ref[...]
    h = jnp.maximum(h, 0.0)
    rw = jnp.dot(h, w2w_ref[...], preferred_element_type=f32, precision=lax.Precision.HIGHEST) + b2w_ref[...]
    rh = jnp.dot(h, w2h_ref[...], preferred_element_type=f32, precision=lax.Precision.HIGHEST) + b2h_ref[...]
    rd = jnp.dot(h, w2d_ref[...], preferred_element_type=f32, precision=lax.Precision.HIGHEST) + b2d_ref[...]

    bb = bb_ref[...]
    p = p_ref[...]
    u = u_ref[...]

    ew = jnp.exp(rw - jnp.max(rw, axis=1, keepdims=True))
    eh = jnp.exp(rh - jnp.max(rh, axis=1, keepdims=True))
    denw = jnp.dot(jnp.dot(ew, p, preferred_element_type=f32, precision=lax.Precision.HIGHEST), bb,
                   preferred_element_type=f32, precision=lax.Precision.HIGHEST)
    denh = jnp.dot(jnp.dot(eh, p, preferred_element_type=f32, precision=lax.Precision.HIGHEST), bb,
                   preferred_element_type=f32, precision=lax.Precision.HIGHEST)
    widths = _SPAN * (_MINW + (1.0 - _MINW * _NUM_BINS) * ew / denw)
    heights = _SPAN * (_MINH + (1.0 - _MINH * _NUM_BINS) * eh / denh)
    derivs = _softplus(rd + _DCONST) + _MIND

    kf = lax.broadcasted_iota(jnp.int32, rw.shape, 1)
    kf = (kf % _NUM_BINS).astype(f32)
    derivs = jnp.where(kf == float(_NUM_BINS - 1), 1.0, derivs)

    cumw = jnp.dot(widths, u, preferred_element_type=f32, precision=lax.Precision.HIGHEST)
    cumh = jnp.dot(heights, u, preferred_element_type=f32, precision=lax.Precision.HIGHEST)

    out_mask = (x1 <= _BLIM) | (x1 >= _ULIM)
    x_in = jnp.where(out_mask, 0.0, x1)
    xb = jnp.dot(x_in, bb, preferred_element_type=f32, precision=lax.Precision.HIGHEST)

    ind = jnp.where(_LEFT + cumw <= xb, 1.0, 0.0)
    cnt = jnp.dot(ind, p, preferred_element_type=f32, precision=lax.Precision.HIGHEST)
    binf = jnp.minimum(cnt, float(_NUM_BINS - 1))
    binb = jnp.dot(binf, bb, preferred_element_type=f32, precision=lax.Precision.HIGHEST)
    oh = jnp.where(kf == binb, 1.0, 0.0)
    ohm = jnp.where(kf == binb - 1.0, 1.0, 0.0)

    wk = jnp.dot(widths * oh, p, preferred_element_type=f32, precision=lax.Precision.HIGHEST)
    hk = jnp.dot(heights * oh, p, preferred_element_type=f32, precision=lax.Precision.HIGHEST)
    xk = _LEFT + jnp.dot(cumw * ohm, p, preferred_element_type=f32, precision=lax.Precision.HIGHEST)
    yk = _LEFT + jnp.dot(cumh * ohm, p, preferred_element_type=f32, precision=lax.Precision.HIGHEST)
    dk = (jnp.dot(derivs * ohm, p, preferred_element_type=f32, precision=lax.Precision.HIGHEST)
          + jnp.where(binf == 0.0, 1.0, 0.0))
    dk1 = jnp.dot(derivs * oh, p, preferred_element_type=f32, precision=lax.Precision.HIGHEST)

    sk = hk / wk
    eps = (x_in - xk) / wk
    eps_term = eps * (1.0 - eps)
    eps2 = eps * eps
    beta = sk + (dk1 + dk - 2.0 * sk) * eps_term
    alpha = hk * (sk * eps2 + dk * eps_term)
    y1 = jnp.where(out_mask, x1, yk + alpha / beta)

    dxb_arg = dk1 * eps2 + 2.0 * sk * eps_term + dk * (1.0 - eps) * (1.0 - eps)
    ld = 2.0 * jnp.log(sk) + jnp.log(dxb_arg) - 2.0 * jnp.log(beta)
    ld = jnp.where(out_mask, 0.0, ld)

    out_ref[...] = (jnp.dot(y1, e1_ref[...], preferred_element_type=f32, precision=lax.Precision.HIGHEST)
                    + jnp.dot(x2, e2_ref[...], preferred_element_type=f32, precision=lax.Precision.HIGHEST))
    ld_ref[...] = jnp.sum(ld, axis=1, keepdims=True)


def _constants():
    e1t = np.zeros((32, _NCH), np.float32)
    e2t = np.zeros((32, _NCH), np.float32)
    for j in range(_NCH):
        e1t[2 * j, j] = 1.0
        e2t[2 * j + 1, j] = 1.0
    e1 = e1t.T.copy()
    e2 = e2t.T.copy()
    bb = np.zeros((_NCH, _GL), np.float32)
    for j in range(_NCH):
        bb[j, j * _NUM_BINS:(j + 1) * _NUM_BINS] = 1.0
    p = bb.T.copy()
    u = np.zeros((_GL, _GL), np.float32)
    for a in range(_GL):
        for b in range(_GL):
            if a // _NUM_BINS == b // _NUM_BINS and a <= b:
                u[a, b] = 1.0
    return e1t, e2t, e1, e2, bb, p, u


def kernel(x, W0, b0, W1, b1, W2, b2):
    batch = x.shape[0]
    bt = min(1024, batch)
    grid = batch // bt
    mlp_dim = W0.shape[1]

    w2r = W2.reshape(mlp_dim, _NCH, 3 * _NUM_BINS - 1)
    w2w = w2r[:, :, :_NUM_BINS].reshape(mlp_dim, _GL)
    w2h = w2r[:, :, _NUM_BINS:2 * _NUM_BINS].reshape(mlp_dim, _GL)
    w2d = jnp.pad(w2r[:, :, 2 * _NUM_BINS:],
                  ((0, 0), (0, 0), (0, 1))).reshape(mlp_dim, _GL)
    b2r = b2.reshape(_NCH, 3 * _NUM_BINS - 1)
    b2w = b2r[:, :_NUM_BINS].reshape(1, _GL)
    b2h = b2r[:, _NUM_BINS:2 * _NUM_BINS].reshape(1, _GL)
    b2d = jnp.pad(b2r[:, 2 * _NUM_BINS:], ((0, 0), (0, 1))).reshape(1, _GL)

    e1t, e2t, e1, e2, bb, p, u = _constants()

    def rep(shape):
        return pl.BlockSpec(shape, lambda i: (0,) * len(shape))

    out, ld = pl.pallas_call(
        _spline_body,
        grid=(grid,),
        in_specs=[
            pl.BlockSpec((bt, 32), lambda i: (i, 0)),
            rep(W0.shape), rep((1, mlp_dim)),
            rep(W1.shape), rep((1, mlp_dim)),
            rep((mlp_dim, _GL)), rep((1, _GL)),
            rep((mlp_dim, _GL)), rep((1, _GL)),
            rep((mlp_dim, _GL)), rep((1, _GL)),
            rep(e1t.shape), rep(e2t.shape), rep(e1.shape), rep(e2.shape),
            rep(bb.shape), rep(u.shape), rep(p.shape),
        ],
        out_specs=[
            pl.BlockSpec((bt, 32), lambda i: (i, 0)),
            pl.BlockSpec((bt, 1), lambda i: (i, 0)),
        ],
        out_shape=[
            jax.ShapeDtypeStruct((batch, 32), jnp.float32),
            jax.ShapeDtypeStruct((batch, 1), jnp.float32),
        ],
        compiler_params=pltpu.CompilerParams(
            dimension_semantics=("arbitrary",)),
    )(x, W0, b0.reshape(1, mlp_dim), W1, b1.reshape(1, mlp_dim),
      w2w, b2w, w2h, b2h, w2d, b2d,
      jnp.asarray(e1t), jnp.asarray(e2t), jnp.asarray(e1), jnp.asarray(e2),
      jnp.asarray(bb), jnp.asarray(u), jnp.asarray(p))
    return out, ld.reshape(batch)


# speedup vs baseline: 3.4669x; 3.4330x over previous
import math

import jax
import jax.numpy as jnp
import numpy as np
from jax import lax
from jax.experimental import pallas as pl
from jax.experimental.pallas import tpu as pltpu

_NUM_BINS = 8
_NCH = 16
_GL = _NCH * _NUM_BINS
_LEFT = -1.0
_SPAN = 2.0
_MINW = 1e-4
_MIND = 1e-4
_BLIM = _LEFT + 1e-3
_ULIM = -_LEFT - 1e-3
_DCONST = math.log(math.exp(1.0 - _MIND) - 1.0)
_PD = lax.Precision.DEFAULT


def _softplus(z):
    return jnp.maximum(z, 0.0) + jnp.log(1.0 + jnp.exp(-jnp.abs(z)))


def _dot01(a, m):
    hi = a.astype(jnp.bfloat16).astype(jnp.float32)
    lo = a - hi
    return (jnp.dot(hi, m, preferred_element_type=jnp.float32, precision=_PD)
            + jnp.dot(lo, m, preferred_element_type=jnp.float32, precision=_PD))


def _spline_body(x_ref, w0_ref, b0_ref, w1_ref, b1_ref, w2_ref, b2_ref,
                 e12t_ref, ee_ref, bb_ref, pp2_ref, bb2_ref, uu_ref,
                 sel_ref, out_ref, ld_ref):
    f32 = jnp.float32
    x = x_ref[...]
    xe = _dot01(x, e12t_ref[...])
    x1 = xe[:, :_NCH]
    x2 = xe[:, _NCH:]

    h = jnp.dot(x2, w0_ref[...], preferred_element_type=f32,
                precision=_PD) + b0_ref[...]
    h = jnp.maximum(h, 0.0)
    h = jnp.dot(h, w1_ref[...], preferred_element_type=f32,
                precision=_PD) + b1_ref[...]
    h = jnp.maximum(h, 0.0)
    rwhd = jnp.dot(h, w2_ref[...], preferred_element_type=f32,
                   precision=_PD) + b2_ref[...]
    rwh = rwhd[:, :2 * _GL]
    rd = rwhd[:, 2 * _GL:]

    ewh = jnp.exp(rwh - jnp.max(rwh, axis=1, keepdims=True))
    den = _dot01(ewh, pp2_ref[...])
    den_b = _dot01(den, bb2_ref[...])
    wh = _SPAN * (_MINW + (1.0 - _MINW * _NUM_BINS) * ewh / den_b)
    cumwh = _dot01(wh, uu_ref[...])
    widths = wh[:, :_GL]
    heights = wh[:, _GL:]
    cumw = cumwh[:, :_GL]
    cumh = cumwh[:, _GL:]

    kf = lax.broadcasted_iota(jnp.int32, widths.shape, 1)
    kf = (kf % _NUM_BINS).astype(f32)
    derivs = _softplus(rd + _DCONST) + _MIND
    derivs = jnp.where(kf == float(_NUM_BINS - 1), 1.0, derivs)

    out_mask = (x1 <= _BLIM) | (x1 >= _ULIM)
    x_in = jnp.where(out_mask, 0.0, x1)
    bb = bb_ref[...]
    xb = _dot01(x_in, bb)
    mask_b = jnp.dot(jnp.where(out_mask, 1.0, 0.0), bb,
                     preferred_element_type=f32, precision=_PD)

    edges = _LEFT + cumw
    ind = jnp.where(edges <= xb, 1.0, 0.0)
    indp = pltpu.roll(ind, 1, 1)
    indp = jnp.where(kf == 0.0, 1.0, indp)
    oh = indp * jnp.where(kf == float(_NUM_BINS - 1), 1.0, 1.0 - ind)

    xk = edges - widths
    yk = _LEFT + cumh - heights
    dk = jnp.where(kf == 0.0, 1.0, pltpu.roll(derivs, 1, 1))
    dk1 = derivs
    sk = heights / widths
    eps = jnp.clip((xb - xk) / widths, 0.0, 1.0)
    eps_term = eps * (1.0 - eps)
    eps2 = eps * eps
    beta = sk + (dk1 + dk - 2.0 * sk) * eps_term
    alpha = heights * (sk * eps2 + dk * eps_term)
    y_cand = yk + alpha / beta
    dxb_arg = dk1 * eps2 + 2.0 * sk * eps_term + dk * (1.0 - eps) * (1.0 - eps)
    ld_cand = 2.0 * jnp.log(sk) + jnp.log(dxb_arg) - 2.0 * jnp.log(beta)

    cat = jnp.concatenate([y_cand * oh, ld_cand * (oh * (1.0 - mask_b))],
                          axis=1)
    res = _dot01(cat, sel_ref[...])
    y1 = jnp.where(out_mask, x1, res[:, :_NCH])

    out_ref[...] = _dot01(jnp.concatenate([y1, x2], axis=1), ee_ref[...])
    ld_ref[...] = res[:, _NCH:_NCH + 1]


def _constants():
    e12t = np.zeros((32, 32), np.float32)
    ee = np.zeros((32, 32), np.float32)
    for j in range(_NCH):
        e12t[2 * j, j] = 1.0
        e12t[2 * j + 1, _NCH + j] = 1.0
        ee[j, 2 * j] = 1.0
        ee[_NCH + j, 2 * j + 1] = 1.0
    bb = np.zeros((_NCH, _GL), np.float32)
    for j in range(_NCH):
        bb[j, j * _NUM_BINS:(j + 1) * _NUM_BINS] = 1.0
    p = bb.T.copy()
    u = np.zeros((_GL, _GL), np.float32)
    for a in range(_GL):
        for b in range(_GL):
            if a // _NUM_BINS == b // _NUM_BINS and a <= b:
                u[a, b] = 1.0
    pp2 = np.zeros((2 * _GL, 32), np.float32)
    pp2[:_GL, :_NCH] = p
    pp2[_GL:, _NCH:] = p
    bb2 = np.zeros((32, 2 * _GL), np.float32)
    bb2[:_NCH, :_GL] = bb
    bb2[_NCH:, _GL:] = bb
    uu = np.zeros((2 * _GL, 2 * _GL), np.float32)
    uu[:_GL, :_GL] = u
    uu[_GL:, _GL:] = u
    sel = np.zeros((2 * _GL, 32), np.float32)
    sel[:_GL, :_NCH] = p
    sel[_GL:, _NCH] = 1.0
    return e12t, ee, bb, pp2, bb2, uu, sel


def kernel(x, W0, b0, W1, b1, W2, b2):
    batch = x.shape[0]
    bt = min(1024, batch)
    grid = batch // bt
    mlp_dim = W0.shape[1]

    w2r = W2.reshape(mlp_dim, _NCH, 3 * _NUM_BINS - 1)
    w2w = w2r[:, :, :_NUM_BINS].reshape(mlp_dim, _GL)
    w2h = w2r[:, :, _NUM_BINS:2 * _NUM_BINS].reshape(mlp_dim, _GL)
    w2d = jnp.pad(w2r[:, :, 2 * _NUM_BINS:],
                  ((0, 0), (0, 0), (0, 1))).reshape(mlp_dim, _GL)
    w2all = jnp.concatenate([w2w, w2h, w2d], axis=1)
    b2r = b2.reshape(_NCH, 3 * _NUM_BINS - 1)
    b2w = b2r[:, :_NUM_BINS].reshape(1, _GL)
    b2h = b2r[:, _NUM_BINS:2 * _NUM_BINS].reshape(1, _GL)
    b2d = jnp.pad(b2r[:, 2 * _NUM_BINS:], ((0, 0), (0, 1))).reshape(1, _GL)
    b2all = jnp.concatenate([b2w, b2h, b2d], axis=1)

    e12t, ee, bb, pp2, bb2, uu, sel = _constants()

    def rep(shape):
        return pl.BlockSpec(shape, lambda i: (0,) * len(shape))

    out, ld = pl.pallas_call(
        _spline_body,
        grid=(grid,),
        in_specs=[
            pl.BlockSpec((bt, 32), lambda i: (i, 0)),
            rep(W0.shape), rep((1, mlp_dim)),
            rep(W1.shape), rep((1, mlp_dim)),
            rep((mlp_dim, 3 * _GL)), rep((1, 3 * _GL)),
            rep(e12t.shape), rep(ee.shape), rep(bb.shape),
            rep(pp2.shape), rep(bb2.shape), rep(uu.shape), rep(sel.shape),
        ],
        out_specs=[
            pl.BlockSpec((bt, 32), lambda i: (i, 0)),
            pl.BlockSpec((bt, 1), lambda i: (i, 0)),
        ],
        out_shape=[
            jax.ShapeDtypeStruct((batch, 32), jnp.float32),
            jax.ShapeDtypeStruct((batch, 1), jnp.float32),
        ],
        compiler_params=pltpu.CompilerParams(
            dimension_semantics=("arbitrary",)),
    )(x, W0, b0.reshape(1, mlp_dim), W1, b1.reshape(1, mlp_dim),
      w2all, b2all,
      jnp.asarray(e12t), jnp.asarray(ee), jnp.asarray(bb),
      jnp.asarray(pp2), jnp.asarray(bb2), jnp.asarray(uu), jnp.asarray(sel))
    return out, ld.reshape(batch)
